# baseline (speedup 1.0000x reference)
import jax
import jax.numpy as jnp
from jax import lax
from jax.experimental import pallas as pl
from jax.experimental.pallas import tpu as pltpu
from jax.experimental.pallas import tpu_sc as plsc

_N_DENSE = 13
_N_SPARSE = 26
_VOCAB = 100000
_DIM = 32
_B = 16384
_ROW = _N_DENSE + _N_SPARSE * _DIM

_NC = 2
_NS = 16
_NW = _NC * _NS
_RPW = _B // _NW
_M = 64
_NCHUNK = _RPW // _M
_IDX_PER_CHUNK = _M * _N_SPARSE
_IDX_ROWS = _IDX_PER_CHUNK // 128
_XW = _M * (_N_DENSE + _N_SPARSE)
_OW = _M * _ROW


def _body(x_hbm, tab_hbm, out_hbm, x_v, ridx_v, emb_v, out_v, sem):
    wid = lax.axis_index("s") * _NC + lax.axis_index("c")
    iot = lax.iota(jnp.int32, 16)
    mask10 = iot < (_N_SPARSE - 16)
    mask13 = iot < _N_DENSE

    def chunk_body(c, carry):
        b0 = wid * _RPW + c * _M

        pltpu.sync_copy(x_hbm.at[pl.ds(b0 * 39, _XW)], x_v.at[pl.ds(0, _XW)])

        def idx_body(r, c2):
            base = r * 39 + _N_DENSE
            xv0 = plsc.load_gather(x_v, [base + iot])
            gi0 = xv0.astype(jnp.int32) + iot * _VOCAB
            p0 = r * _N_SPARSE + iot
            plsc.store_scatter(ridx_v, [p0 >> 7, p0 & 127], gi0)
            xv1 = plsc.load_gather(x_v, [base + 16 + iot], mask=mask10)
            gi1 = xv1.astype(jnp.int32) + (iot + 16) * _VOCAB
            p1 = p0 + 16
            plsc.store_scatter(ridx_v, [p1 >> 7, p1 & 127], gi1, mask=mask10)
            return c2

        lax.fori_loop(0, _M, idx_body, 0)

        copies = [
            pltpu.async_copy(
                tab_hbm.at[ridx_v.at[j]],
                emb_v.at[pl.ds(j * 128, 128)],
                sem,
            )
            for j in range(_IDX_ROWS)
        ]
        for cp in copies:
            cp.wait()

        def row_body(r, c2):
            obase = r * _ROW
            dv = plsc.load_gather(x_v, [r * 39 + iot], mask=mask13)
            plsc.store_scatter(out_v, [obase + iot], dv, mask=mask13)
            erow0 = r * _N_SPARSE
            for s in range(2 * _N_SPARSE):
                row = jnp.full((16,), 0, jnp.int32) + (erow0 + (s >> 1))
                ev = plsc.load_gather(emb_v, [row, (s & 1) * 16 + iot])
                plsc.store_scatter(out_v, [obase + _N_DENSE + s * 16 + iot], ev)
            return c2

        lax.fori_loop(0, _M, row_body, 0)

        pltpu.sync_copy(out_v, out_hbm.at[pl.ds(b0 * _ROW, _OW)])
        return carry

    lax.fori_loop(0, _NCHUNK, chunk_body, 0)


@jax.jit
def kernel(x, tables):
    x_flat = x.reshape(-1)
    tab_flat = tables.reshape(_N_SPARSE * _VOCAB, _DIM)
    mesh = plsc.VectorSubcoreMesh(core_axis_name="c", subcore_axis_name="s")
    out_flat = pl.kernel(
        _body,
        mesh=mesh,
        compiler_params=pltpu.CompilerParams(
            needs_layout_passes=False, use_tc_tiling_on_sc=False
        ),
        out_type=jax.ShapeDtypeStruct((_B * _ROW,), jnp.float32),
        scratch_types=[
            pltpu.VMEM((_XW + 16,), jnp.float32),
            pltpu.VMEM((_IDX_ROWS, 128), jnp.int32),
            pltpu.VMEM((_IDX_PER_CHUNK, _DIM), jnp.float32),
            pltpu.VMEM((_OW,), jnp.float32),
            pltpu.SemaphoreType.DMA,
        ],
    )(x_flat, tab_flat)
    return out_flat.reshape(_B, _ROW)
